# baseline (speedup 1.0000x reference)
import functools

import numpy as np
import jax
import jax.numpy as jnp
from jax import lax
from jax.experimental import pallas as pl
from jax.experimental.pallas import tpu as pltpu
from jax.experimental.pallas import tpu_sc as plsc

HIDDEN = 1024
HHALF = HIDDEN // 2
TABLE_SIZE = 16384
NUM_TABLES = 4
NUM_LOOKUPS = 8
EPS = 1e-6
SCALE = 1.0 / (HIDDEN ** 0.5)

NUM_CORES = 2
NUM_SUBCORES = 16
NW = NUM_CORES * NUM_SUBCORES
CHUNK = 32
LANES = 16


def _build_sc_gather(B, S):
    TOK = B * S
    TPW = TOK // NW
    SPB = S // TPW
    PADS = S + 8
    ROWS_PW = TPW * NUM_LOOKUPS
    NCHUNK = ROWS_PW // CHUNK
    CPS = TPW // CHUNK

    mesh = plsc.VectorSubcoreMesh(core_axis_name="c", subcore_axis_name="s")

    @functools.partial(
        pl.kernel,
        mesh=mesh,
        out_type=jax.ShapeDtypeStruct((NUM_LOOKUPS * TOK, HHALF), jnp.int32),
        scratch_types=[
            pltpu.VMEM((TPW + 8,), jnp.int32),
            pltpu.VMEM((12, LANES), jnp.int32),
            pltpu.VMEM((ROWS_PW,), jnp.int32),
            pltpu.VMEM((CHUNK, HIDDEN), jnp.float32),
            pltpu.VMEM((CHUNK, HIDDEN), jnp.float32),
            pltpu.VMEM((CHUNK, HIDDEN), jnp.float32),
            pltpu.VMEM((CHUNK // 2, HHALF), jnp.int32),
            pltpu.VMEM((CHUNK // 2, HHALF), jnp.int32),
            pltpu.SemaphoreType.DMA,
            pltpu.SemaphoreType.DMA,
            pltpu.SemaphoreType.DMA,
            pltpu.SemaphoreType.DMA,
            pltpu.SemaphoreType.DMA,
        ],
    )
    def sc_gather(tok_hbm, coef_hbm, table_hbm, eng_hbm,
                  tkn_v, coef_v, idx_v, fb0, fb1, fb2, pba, pbb,
                  sg0, sg1, sg2, soa, sob):
        w = lax.axis_index("s") * NUM_CORES + lax.axis_index("c")
        w = w.astype(jnp.int32)
        spb = jnp.int32(SPB)
        bb = lax.div(w, spb)
        s0 = lax.rem(w, spb) * TPW
        off = bb * PADS + s0

        pltpu.sync_copy(tok_hbm.at[pl.ds(off, TPW + 8)], tkn_v)
        pltpu.sync_copy(coef_hbm, coef_v)

        csp = [[coef_v[k * 3 + j, :] for j in range(3)]
               for k in range(NUM_TABLES)]
        mvec = jnp.full((LANES,), TABLE_SIZE, jnp.int32)

        def hash_body(g, carry):
            base = g * LANES
            t0 = tkn_v[pl.ds(8 + base, LANES)]
            t1 = tkn_v[pl.ds(7 + base, LANES)]
            t2 = tkn_v[pl.ds(6 + base, LANES)]
            m0 = lax.rem(t0, mvec)
            m1 = lax.rem(t1, mvec)
            m2 = lax.rem(t2, mvec)
            for kk in range(NUM_LOOKUPS):
                if kk < NUM_TABLES:
                    k = kk
                    h = lax.rem(csp[k][0] * m1 + csp[k][1] * m0, mvec)
                else:
                    k = kk - NUM_TABLES
                    h = lax.rem(csp[k][0] * m2 + csp[k][1] * m1 + csp[k][2] * m0, mvec)
                idx_v[pl.ds(kk * TPW + base, LANES)] = h + kk * TABLE_SIZE
            return carry

        lax.fori_loop(jnp.int32(0), jnp.int32(TPW // LANES), hash_body,
                      jnp.int32(0))

        fbufs = (fb0, fb1, fb2)
        gsems = (sg0, sg1, sg2)

        def g_copy(it, p):
            it = jnp.asarray(it, jnp.int32)
            return pltpu.make_async_copy(
                table_hbm.at[idx_v.at[pl.ds(it * CHUNK, CHUNK)]],
                fbufs[p], gsems[p])

        HALF = CHUNK // 2
        pbufs = (pba, pbb)
        psems = (soa, sob)

        def o_copy(it, half):
            it = jnp.asarray(it, jnp.int32)
            cps = jnp.int32(CPS)
            kk = lax.div(it, cps)
            c = lax.rem(it, cps)
            base_row = kk * TOK + w * TPW + c * CHUNK + half * HALF
            return pltpu.make_async_copy(
                pbufs[half], eng_hbm.at[pl.ds(base_row, HALF)], psems[half])

        rnd = jnp.full((LANES,), 0x8000, jnp.int32)
        hi_mask = jnp.full((LANES,), -65536, jnp.int32)
        sixteen = jnp.full((LANES,), 16, jnp.int32)

        def pack_half(p, half):
            fb = fbufs[p]
            pb = pbufs[half]
            r0 = half * HALF

            @plsc.parallel_loop(jnp.int32(0), jnp.int32(HALF),
                                step=jnp.int32(1), unroll=2)
            def row_body(r):
                for v in range(HHALF // LANES):
                    a = fb[r0 + r, pl.ds(v * LANES, LANES)]
                    b = fb[r0 + r, pl.ds(HHALF + v * LANES, LANES)]
                    ia = lax.bitcast_convert_type(a, jnp.int32) + rnd
                    ib = lax.bitcast_convert_type(b, jnp.int32) + rnd
                    word = lax.shift_right_logical(ia, sixteen) | (ib & hi_mask)
                    pb[r, pl.ds(v * LANES, LANES)] = word

        g_copy(0, 0).start()
        g_copy(1, 1).start()

        def step(it, p):
            q = (p + 2) % 3
            g_copy(it, p).wait()

            @pl.when(it + 2 <= NCHUNK - 1)
            def _():
                g_copy(it + 2, q).start()

            @pl.when(it >= 1)
            def _():
                o_copy(it - 1, 0).wait()

            pack_half(p, 0)
            o_copy(it, 0).start()

            @pl.when(it >= 1)
            def _():
                o_copy(it - 1, 1).wait()

            pack_half(p, 1)
            o_copy(it, 1).start()

        def pipe(kq, carry):
            it = kq * 3
            step(it, 0)
            step(it + 1, 1)
            step(it + 2, 2)
            return carry

        lax.fori_loop(jnp.int32(0), jnp.int32(NCHUNK // 3), pipe, jnp.int32(0))
        step(NCHUNK - 1, (NCHUNK - 1) % 3)
        o_copy(NCHUNK - 1, 0).wait()
        o_copy(NCHUNK - 1, 1).wait()

    return sc_gather


def _build_tc_fuse(TOK):
    TB = 256
    grid = (TOK // TB,)

    def body(h_ref, e_ref, nh_ref, ne_ref, b_ref, o_ref):
        h = h_ref[...]
        e32 = e_ref[...]
        ea = lax.bitcast_convert_type(e32 << 16, jnp.float32)
        eb = lax.bitcast_convert_type(e32 & jnp.int32(-65536), jnp.float32)
        wprod = nh_ref[...] * ne_ref[...]
        var_h = jnp.mean(h * h, axis=-1, keepdims=True)
        a_t = lax.rsqrt(var_h + EPS) * SCALE
        q = h * wprod
        qa = q[:, :HHALF]
        qb = q[:, HHALF:]
        dots = (jnp.sum(qa[None, :, :] * ea, axis=-1)
                + jnp.sum(qb[None, :, :] * eb, axis=-1))
        sqs = (jnp.sum(ea * ea, axis=-1)
               + jnp.sum(eb * eb, axis=-1)) * (1.0 / HIDDEN)
        score = a_t[None, :, 0] * lax.rsqrt(sqs + EPS) * dots + b_ref[0, 0]
        alpha = jax.nn.sigmoid(score)
        ca = jnp.sum(alpha[:, :, None] * ea, axis=0)
        cb = jnp.sum(alpha[:, :, None] * eb, axis=0)
        o_ref[...] = h + jnp.concatenate([ca, cb], axis=-1)

    z = np.int32(0)
    return pl.pallas_call(
        body,
        grid=grid,
        in_specs=[
            pl.BlockSpec((TB, HIDDEN), lambda i: (i, z)),
            pl.BlockSpec((NUM_LOOKUPS, TB, HHALF), lambda i: (z, i, z)),
            pl.BlockSpec((1, HIDDEN), lambda i: (z, z)),
            pl.BlockSpec((1, HIDDEN), lambda i: (z, z)),
            pl.BlockSpec((1, 1), lambda i: (z, z)),
        ],
        out_specs=pl.BlockSpec((TB, HIDDEN), lambda i: (i, z)),
        out_shape=jax.ShapeDtypeStruct((TOK, HIDDEN), jnp.float32),
    )


def kernel(token_ids, hidden_states, table, coeffs, norm_h_w, norm_e_w, bias):
    B, S = token_ids.shape
    TOK = B * S
    tok_flat = jnp.pad(token_ids.astype(jnp.int32), ((0, 0), (8, 0))).reshape(-1)
    coef_splat = jnp.broadcast_to(
        coeffs.astype(jnp.int32).reshape(-1)[:, None], (12, LANES))

    eng = _build_sc_gather(B, S)(tok_flat, coef_splat, table.astype(jnp.float32))
    eng3 = eng.reshape(NUM_LOOKUPS, TOK, HHALF)

    out = _build_tc_fuse(TOK)(
        hidden_states.reshape(TOK, HIDDEN),
        eng3,
        norm_h_w.reshape(1, HIDDEN).astype(jnp.float32),
        norm_e_w.reshape(1, HIDDEN).astype(jnp.float32),
        bias.reshape(1, 1).astype(jnp.float32),
    )
    return out.reshape(B, S, HIDDEN)


# speedup vs baseline: 1.3543x; 1.3543x over previous
import functools

import numpy as np
import jax
import jax.numpy as jnp
from jax import lax
from jax.experimental import pallas as pl
from jax.experimental.pallas import tpu as pltpu
from jax.experimental.pallas import tpu_sc as plsc

HIDDEN = 1024
HHALF = HIDDEN // 2
TABLE_SIZE = 16384
NUM_TABLES = 4
NUM_LOOKUPS = 8
EPS = 1e-6
SCALE = 1.0 / (HIDDEN ** 0.5)

NUM_CORES = 2
NUM_SUBCORES = 16
NW = NUM_CORES * NUM_SUBCORES
CHUNK = 32
LANES = 16


def _build_sc_gather(B, S):
    TOK = B * S
    TPW = TOK // NW
    SPB = S // TPW
    PADS = S + 8
    ROWS_PW = TPW * NUM_LOOKUPS
    NCHUNK = ROWS_PW // CHUNK
    CPS = TPW // CHUNK

    mesh = plsc.VectorSubcoreMesh(core_axis_name="c", subcore_axis_name="s")

    @functools.partial(
        pl.kernel,
        mesh=mesh,
        out_type=jax.ShapeDtypeStruct((NUM_LOOKUPS * TOK, HHALF), jnp.int32),
        scratch_types=[
            pltpu.VMEM((TPW + 8,), jnp.int32),
            pltpu.VMEM((12, LANES), jnp.int32),
            pltpu.VMEM((ROWS_PW,), jnp.int32),
            pltpu.VMEM((CHUNK, HIDDEN), jnp.float32),
            pltpu.VMEM((CHUNK, HIDDEN), jnp.float32),
            pltpu.VMEM((CHUNK, HHALF), jnp.int32),
            pltpu.VMEM((CHUNK, HHALF), jnp.int32),
            pltpu.SemaphoreType.DMA,
            pltpu.SemaphoreType.DMA,
            pltpu.SemaphoreType.DMA,
        ],
    )
    def sc_gather(tok_hbm, coef_hbm, table_hbm, eng_hbm,
                  tkn_v, coef_v, idx_v, fb0, fb1, pb0, pb1, sg, so0, so1):
        w = lax.axis_index("s") * NUM_CORES + lax.axis_index("c")
        w = w.astype(jnp.int32)
        spb = jnp.int32(SPB)
        bb = lax.div(w, spb)
        s0 = lax.rem(w, spb) * TPW
        off = bb * PADS + s0

        pltpu.sync_copy(tok_hbm.at[pl.ds(off, TPW + 8)], tkn_v)
        pltpu.sync_copy(coef_hbm, coef_v)

        csp = [[coef_v[k * 3 + j, :] for j in range(3)]
               for k in range(NUM_TABLES)]
        mvec = jnp.full((LANES,), TABLE_SIZE, jnp.int32)

        def hash_body(g, carry):
            base = g * LANES
            t0 = tkn_v[pl.ds(8 + base, LANES)]
            t1 = tkn_v[pl.ds(7 + base, LANES)]
            t2 = tkn_v[pl.ds(6 + base, LANES)]
            m0 = lax.rem(t0, mvec)
            m1 = lax.rem(t1, mvec)
            m2 = lax.rem(t2, mvec)
            for kk in range(NUM_LOOKUPS):
                if kk < NUM_TABLES:
                    k = kk
                    h = lax.rem(csp[k][0] * m1 + csp[k][1] * m0, mvec)
                else:
                    k = kk - NUM_TABLES
                    h = lax.rem(csp[k][0] * m2 + csp[k][1] * m1 + csp[k][2] * m0, mvec)
                idx_v[pl.ds(kk * TPW + base, LANES)] = h + kk * TABLE_SIZE
            return carry

        lax.fori_loop(jnp.int32(0), jnp.int32(TPW // LANES), hash_body,
                      jnp.int32(0))

        fbufs = (fb0, fb1)
        pbufs = (pb0, pb1)
        osems = (so0, so1)

        def g_copy(it, p):
            it = jnp.asarray(it, jnp.int32)
            return pltpu.make_async_copy(
                table_hbm.at[idx_v.at[pl.ds(it * CHUNK, CHUNK)]],
                fbufs[p], sg)

        def o_copy(it, p):
            it = jnp.asarray(it, jnp.int32)
            cps = jnp.int32(CPS)
            kk = lax.div(it, cps)
            c = lax.rem(it, cps)
            base_row = kk * TOK + w * TPW + c * CHUNK
            return pltpu.make_async_copy(
                pbufs[p], eng_hbm.at[pl.ds(base_row, CHUNK)], osems[p])

        rnd = jnp.full((LANES,), 0x8000, jnp.int32)
        hi_mask = jnp.full((LANES,), -65536, jnp.int32)
        sixteen = jnp.full((LANES,), 16, jnp.int32)

        def pack_chunk(p):
            fb = fbufs[p]
            pb = pbufs[p]

            @plsc.parallel_loop(jnp.int32(0), jnp.int32(CHUNK),
                                step=jnp.int32(1), unroll=2)
            def row_body(r):
                for v in range(HHALF // LANES):
                    a = fb[r, pl.ds(v * LANES, LANES)]
                    b = fb[r, pl.ds(HHALF + v * LANES, LANES)]
                    ia = lax.bitcast_convert_type(a, jnp.int32) + rnd
                    ib = lax.bitcast_convert_type(b, jnp.int32) + rnd
                    word = lax.shift_right_logical(ia, sixteen) | (ib & hi_mask)
                    pb[r, pl.ds(v * LANES, LANES)] = word

        g_copy(0, 0).start()

        def step(it, p):
            g_copy(it, p).wait()

            @pl.when(it + 1 <= NCHUNK - 1)
            def _():
                g_copy(it + 1, 1 - p).start()

            @pl.when(it >= 2)
            def _():
                o_copy(it - 2, p).wait()

            pack_chunk(p)
            o_copy(it, p).start()

        def pipe(kq, carry):
            it = kq * 2
            step(it, 0)
            step(it + 1, 1)
            return carry

        lax.fori_loop(jnp.int32(0), jnp.int32(NCHUNK // 2), pipe, jnp.int32(0))
        o_copy(NCHUNK - 2, 0).wait()
        o_copy(NCHUNK - 1, 1).wait()

    return sc_gather


def _build_tc_fuse(TOK):
    TB = 256
    grid = (TOK // TB,)

    def body(h_ref, e_ref, nh_ref, ne_ref, b_ref, o_ref):
        h = h_ref[...]
        e32 = e_ref[...]
        ea = lax.bitcast_convert_type(e32 << 16, jnp.float32)
        eb = lax.bitcast_convert_type(e32 & jnp.int32(-65536), jnp.float32)
        wprod = nh_ref[...] * ne_ref[...]
        var_h = jnp.mean(h * h, axis=-1, keepdims=True)
        a_t = lax.rsqrt(var_h + EPS) * SCALE
        q = h * wprod
        qa = q[:, :HHALF]
        qb = q[:, HHALF:]
        dots = (jnp.sum(qa[None, :, :] * ea, axis=-1)
                + jnp.sum(qb[None, :, :] * eb, axis=-1))
        sqs = (jnp.sum(ea * ea, axis=-1)
               + jnp.sum(eb * eb, axis=-1)) * (1.0 / HIDDEN)
        score = a_t[None, :, 0] * lax.rsqrt(sqs + EPS) * dots + b_ref[0, 0]
        alpha = jax.nn.sigmoid(score)
        ca = jnp.sum(alpha[:, :, None] * ea, axis=0)
        cb = jnp.sum(alpha[:, :, None] * eb, axis=0)
        o_ref[...] = h + jnp.concatenate([ca, cb], axis=-1)

    z = np.int32(0)
    return pl.pallas_call(
        body,
        grid=grid,
        in_specs=[
            pl.BlockSpec((TB, HIDDEN), lambda i: (i, z)),
            pl.BlockSpec((NUM_LOOKUPS, TB, HHALF), lambda i: (z, i, z)),
            pl.BlockSpec((1, HIDDEN), lambda i: (z, z)),
            pl.BlockSpec((1, HIDDEN), lambda i: (z, z)),
            pl.BlockSpec((1, 1), lambda i: (z, z)),
        ],
        out_specs=pl.BlockSpec((TB, HIDDEN), lambda i: (i, z)),
        out_shape=jax.ShapeDtypeStruct((TOK, HIDDEN), jnp.float32),
    )


def kernel(token_ids, hidden_states, table, coeffs, norm_h_w, norm_e_w, bias):
    B, S = token_ids.shape
    TOK = B * S
    tok_flat = jnp.pad(token_ids.astype(jnp.int32), ((0, 0), (8, 0))).reshape(-1)
    coef_splat = jnp.broadcast_to(
        coeffs.astype(jnp.int32).reshape(-1)[:, None], (12, LANES))

    eng = _build_sc_gather(B, S)(tok_flat, coef_splat, table.astype(jnp.float32))
    eng3 = eng.reshape(NUM_LOOKUPS, TOK, HHALF)

    out = _build_tc_fuse(TOK)(
        hidden_states.reshape(TOK, HIDDEN),
        eng3,
        norm_h_w.reshape(1, HIDDEN).astype(jnp.float32),
        norm_e_w.reshape(1, HIDDEN).astype(jnp.float32),
        bias.reshape(1, 1).astype(jnp.float32),
    )
    return out.reshape(B, S, HIDDEN)


# speedup vs baseline: 1.4171x; 1.0464x over previous
import functools

import numpy as np
import jax
import jax.numpy as jnp
from jax import lax
from jax.experimental import pallas as pl
from jax.experimental.pallas import tpu as pltpu
from jax.experimental.pallas import tpu_sc as plsc

HIDDEN = 1024
HHALF = HIDDEN // 2
TABLE_SIZE = 16384
NUM_TABLES = 4
NUM_LOOKUPS = 8
EPS = 1e-6
SCALE = 1.0 / (HIDDEN ** 0.5)

NUM_CORES = 2
NUM_SUBCORES = 16
NW = NUM_CORES * NUM_SUBCORES
CHUNK = 32
LANES = 16


def _build_sc_gather(B, S):
    TOK = B * S
    TPW = TOK // NW
    SPB = S // TPW
    PADS = S + 8
    ROWS_PW = TPW * NUM_LOOKUPS
    NCHUNK = ROWS_PW // CHUNK
    CPS = TPW // CHUNK

    mesh = plsc.VectorSubcoreMesh(core_axis_name="c", subcore_axis_name="s")

    @functools.partial(
        pl.kernel,
        mesh=mesh,
        out_type=jax.ShapeDtypeStruct((NUM_LOOKUPS * TOK, HHALF), jnp.int32),
        scratch_types=[
            pltpu.VMEM((TPW + 8,), jnp.int32),
            pltpu.VMEM((12, LANES), jnp.int32),
            pltpu.VMEM((ROWS_PW,), jnp.int32),
            pltpu.VMEM((CHUNK, HIDDEN), jnp.float32),
            pltpu.VMEM((CHUNK, HIDDEN), jnp.float32),
            pltpu.VMEM((CHUNK, HHALF), jnp.int32),
            pltpu.VMEM((CHUNK, HHALF), jnp.int32),
            pltpu.SemaphoreType.DMA,
            pltpu.SemaphoreType.DMA,
            pltpu.SemaphoreType.DMA,
            pltpu.SemaphoreType.DMA,
        ],
    )
    def sc_gather(tok_hbm, coef_hbm, table_hbm, eng_hbm,
                  tkn_v, coef_v, idx_v, fb0, fb1, pb0, pb1,
                  sg0, sg1, so0, so1):
        w = lax.axis_index("s") * NUM_CORES + lax.axis_index("c")
        w = w.astype(jnp.int32)
        spb = jnp.int32(SPB)
        bb = lax.div(w, spb)
        s0 = lax.rem(w, spb) * TPW
        off = bb * PADS + s0

        pltpu.sync_copy(tok_hbm.at[pl.ds(off, TPW + 8)], tkn_v)
        pltpu.sync_copy(coef_hbm, coef_v)

        csp = [[coef_v[k * 3 + j, :] for j in range(3)]
               for k in range(NUM_TABLES)]
        mvec = jnp.full((LANES,), TABLE_SIZE, jnp.int32)

        def hash_body(g, carry):
            base = g * LANES
            t0 = tkn_v[pl.ds(8 + base, LANES)]
            t1 = tkn_v[pl.ds(7 + base, LANES)]
            t2 = tkn_v[pl.ds(6 + base, LANES)]
            m0 = lax.rem(t0, mvec)
            m1 = lax.rem(t1, mvec)
            m2 = lax.rem(t2, mvec)
            for kk in range(NUM_LOOKUPS):
                if kk < NUM_TABLES:
                    k = kk
                    h = lax.rem(csp[k][0] * m1 + csp[k][1] * m0, mvec)
                else:
                    k = kk - NUM_TABLES
                    h = lax.rem(csp[k][0] * m2 + csp[k][1] * m1 + csp[k][2] * m0, mvec)
                idx_v[pl.ds(kk * TPW + base, LANES)] = h + kk * TABLE_SIZE
            return carry

        lax.fori_loop(jnp.int32(0), jnp.int32(TPW // LANES), hash_body,
                      jnp.int32(0))

        fbufs = (fb0, fb1)
        pbufs = (pb0, pb1)
        gsems = (sg0, sg1)
        osems = (so0, so1)

        def g_copy(it, p):
            it = jnp.asarray(it, jnp.int32)
            return pltpu.make_async_copy(
                table_hbm.at[idx_v.at[pl.ds(it * CHUNK, CHUNK)]],
                fbufs[p], gsems[p])

        def o_copy(it, p):
            it = jnp.asarray(it, jnp.int32)
            cps = jnp.int32(CPS)
            kk = lax.div(it, cps)
            c = lax.rem(it, cps)
            base_row = kk * TOK + w * TPW + c * CHUNK
            return pltpu.make_async_copy(
                pbufs[p], eng_hbm.at[pl.ds(base_row, CHUNK)], osems[p])

        rnd = jnp.full((LANES,), 0x8000, jnp.int32)
        hi_mask = jnp.full((LANES,), -65536, jnp.int32)
        sixteen = jnp.full((LANES,), 16, jnp.int32)

        def pack_chunk(p):
            fb = fbufs[p]
            pb = pbufs[p]

            @plsc.parallel_loop(jnp.int32(0), jnp.int32(CHUNK),
                                step=jnp.int32(1), unroll=4)
            def row_body(r):
                for v in range(HHALF // LANES):
                    a = fb[r, pl.ds(v * LANES, LANES)]
                    b = fb[r, pl.ds(HHALF + v * LANES, LANES)]
                    ia = lax.bitcast_convert_type(a, jnp.int32) + rnd
                    ib = lax.bitcast_convert_type(b, jnp.int32) + rnd
                    word = lax.shift_right_logical(ia, sixteen) | (ib & hi_mask)
                    pb[r, pl.ds(v * LANES, LANES)] = word

        g_copy(0, 0).start()

        def step(it, p):
            @pl.when(it + 1 <= NCHUNK - 1)
            def _():
                g_copy(it + 1, 1 - p).start()

            g_copy(it, p).wait()

            @pl.when(it >= 2)
            def _():
                o_copy(it - 2, p).wait()

            pack_chunk(p)
            o_copy(it, p).start()

        def pipe(kq, carry):
            it = kq * 2
            step(it, 0)
            step(it + 1, 1)
            return carry

        lax.fori_loop(jnp.int32(0), jnp.int32(NCHUNK // 2), pipe, jnp.int32(0))
        o_copy(NCHUNK - 2, 0).wait()
        o_copy(NCHUNK - 1, 1).wait()

    return sc_gather


def _build_tc_fuse(TOK):
    TB = 256
    grid = (TOK // TB,)

    def body(h_ref, e_ref, nh_ref, ne_ref, b_ref, o_ref):
        h = h_ref[...]
        e32 = e_ref[...]
        ea = lax.bitcast_convert_type(e32 << 16, jnp.float32)
        eb = lax.bitcast_convert_type(e32 & jnp.int32(-65536), jnp.float32)
        wprod = nh_ref[...] * ne_ref[...]
        var_h = jnp.mean(h * h, axis=-1, keepdims=True)
        a_t = lax.rsqrt(var_h + EPS) * SCALE
        q = h * wprod
        qa = q[:, :HHALF]
        qb = q[:, HHALF:]
        dots = (jnp.sum(qa[None, :, :] * ea, axis=-1)
                + jnp.sum(qb[None, :, :] * eb, axis=-1))
        sqs = (jnp.sum(ea * ea, axis=-1)
               + jnp.sum(eb * eb, axis=-1)) * (1.0 / HIDDEN)
        score = a_t[None, :, 0] * lax.rsqrt(sqs + EPS) * dots + b_ref[0, 0]
        alpha = jax.nn.sigmoid(score)
        ca = jnp.sum(alpha[:, :, None] * ea, axis=0)
        cb = jnp.sum(alpha[:, :, None] * eb, axis=0)
        o_ref[...] = h + jnp.concatenate([ca, cb], axis=-1)

    z = np.int32(0)
    return pl.pallas_call(
        body,
        grid=grid,
        in_specs=[
            pl.BlockSpec((TB, HIDDEN), lambda i: (i, z)),
            pl.BlockSpec((NUM_LOOKUPS, TB, HHALF), lambda i: (z, i, z)),
            pl.BlockSpec((1, HIDDEN), lambda i: (z, z)),
            pl.BlockSpec((1, HIDDEN), lambda i: (z, z)),
            pl.BlockSpec((1, 1), lambda i: (z, z)),
        ],
        out_specs=pl.BlockSpec((TB, HIDDEN), lambda i: (i, z)),
        out_shape=jax.ShapeDtypeStruct((TOK, HIDDEN), jnp.float32),
    )


def kernel(token_ids, hidden_states, table, coeffs, norm_h_w, norm_e_w, bias):
    B, S = token_ids.shape
    TOK = B * S
    tok_flat = jnp.pad(token_ids.astype(jnp.int32), ((0, 0), (8, 0))).reshape(-1)
    coef_splat = jnp.broadcast_to(
        coeffs.astype(jnp.int32).reshape(-1)[:, None], (12, LANES))

    eng = _build_sc_gather(B, S)(tok_flat, coef_splat, table.astype(jnp.float32))
    eng3 = eng.reshape(NUM_LOOKUPS, TOK, HHALF)

    out = _build_tc_fuse(TOK)(
        hidden_states.reshape(TOK, HIDDEN),
        eng3,
        norm_h_w.reshape(1, HIDDEN).astype(jnp.float32),
        norm_e_w.reshape(1, HIDDEN).astype(jnp.float32),
        bias.reshape(1, 1).astype(jnp.float32),
    )
    return out.reshape(B, S, HIDDEN)
